# baseline (speedup 1.0000x reference)
import functools

import jax
import jax.numpy as jnp
from jax import lax
from jax.experimental import pallas as pl
from jax.experimental.pallas import tpu as pltpu
from jax.experimental.pallas import tpu_sc as plsc

_LANE = 128
_CHUNK = 512
_SUB = _CHUNK // _LANE
_NBUF = 3


def _gather_rows(ids2d, table, num_workers):
    n_rows, _ = ids2d.shape
    n = n_rows * _LANE
    _, d = table.shape
    per_w = n // num_workers
    n_chunks = per_w // _CHUNK
    idx_rows = per_w // _LANE

    mesh = plsc.VectorSubcoreMesh(core_axis_name="c", subcore_axis_name="s")

    @functools.partial(
        pl.kernel,
        out_type=jax.ShapeDtypeStruct((n, d), jnp.float32),
        mesh=mesh,
        compiler_params=pltpu.CompilerParams(use_tc_tiling_on_sc=False),
        scratch_types=[
            pltpu.VMEM((idx_rows, _LANE), jnp.int32),
            pltpu.VMEM((_NBUF, _CHUNK, d), jnp.float32),
            pltpu.SemaphoreType.DMA,
            pltpu.SemaphoreType.DMA,
            pltpu.SemaphoreType.DMA,
            pltpu.SemaphoreType.DMA,
            pltpu.SemaphoreType.DMA,
            pltpu.SemaphoreType.DMA,
            pltpu.SemaphoreType.DMA,
        ],
    )
    def grab(ids_hbm, tab_hbm, out_hbm, idx_v, rows_v,
             si, sg0, sg1, sg2, ss0, ss1, ss2):
        gat_sems = [sg0, sg1, sg2]
        st_sems = [ss0, ss1, ss2]

        nc = lax.axis_size("c")
        wid = lax.axis_index("s") * nc + lax.axis_index("c")
        out_base = wid * per_w

        def gather_copies(j, s):
            return [
                pltpu.make_async_copy(
                    tab_hbm.at[idx_v.at[j * _SUB + i]],
                    rows_v.at[s, pl.ds(i * _LANE, _LANE)],
                    gat_sems[s],
                )
                for i in range(_SUB)
            ]

        def store_copy(j, s):
            return pltpu.make_async_copy(
                rows_v.at[s],
                out_hbm.at[pl.ds(out_base + j * _CHUNK, _CHUNK)],
                st_sems[s],
            )

        pltpu.make_async_copy(
            ids_hbm.at[pl.ds(wid * idx_rows, idx_rows)], idx_v, si
        ).start()
        pltpu.make_async_copy(
            ids_hbm.at[pl.ds(wid * idx_rows, idx_rows)], idx_v, si
        ).wait()
        for c in gather_copies(0, 0):
            c.start()
        for c in gather_copies(1, 1):
            c.start()

        def step(j, s, first, tail):
            for c in gather_copies(j, s):
                c.wait()
            store_copy(j, s).start()
            if not tail:
                if not first:
                    store_copy(j - 1, (s + 2) % _NBUF).wait()
                for c in gather_copies(j + 2, (s + 2) % _NBUF):
                    c.start()
            else:
                store_copy(j - 1, (s + 2) % _NBUF).wait()

        step(0, 0, first=True, tail=False)
        step(1, 1, first=False, tail=False)

        @pl.loop(2, n_chunks - 3, step=_NBUF)
        def _(g):
            step(g, 2, first=False, tail=False)
            step(g + 1, 0, first=False, tail=False)
            step(g + 2, 1, first=False, tail=False)

        step(n_chunks - 3, (n_chunks - 3) % _NBUF, first=False, tail=False)
        step(n_chunks - 2, (n_chunks - 2) % _NBUF, first=False, tail=True)
        step(n_chunks - 1, (n_chunks - 1) % _NBUF, first=False, tail=True)
        store_copy(n_chunks - 1, (n_chunks - 1) % _NBUF).wait()

    return grab(ids2d, table)


def kernel(input_ids, word_embeddings):
    b, s = input_ids.shape
    v, d = word_embeddings.shape
    n = b * s
    num_workers = 32
    ids2d = input_ids.astype(jnp.int32).reshape(n // _LANE, _LANE)
    out = _gather_rows(ids2d, word_embeddings.astype(jnp.float32),
                       num_workers)
    return out.reshape(b, s, d)
